# baseline (speedup 1.0000x reference)
import functools

import jax
import jax.numpy as jnp
from jax import lax
from jax.experimental import pallas as pl
from jax.experimental.pallas import tpu as pltpu
from jax.experimental.pallas import tpu_sc as plsc

T, K_MAX, D = 256, 128, 128
L = T * K_MAX
NH, DH = 4, 32
DFF = 256
BLK = 256
NB = L // BLK
NKEY = 128
EPS = 1e-5
NEG = -1e9
NBUF = 3


SC_NC, SC_NS = 2, 16
SC_NW = SC_NC * SC_NS
RPW = L // SC_NW
CH = 128
NCH = RPW // CH

@functools.cache
def _sc_kernels():
    mesh = plsc.VectorSubcoreMesh(core_axis_name="c", subcore_axis_name="s")

    @functools.partial(
        pl.kernel,
        out_type=jax.ShapeDtypeStruct((L, D), jnp.float32),
        mesh=mesh,
        scratch_types=[pltpu.VMEM((CH, D), jnp.float32),
                       pltpu.VMEM((CH, D), jnp.float32),
                       pltpu.VMEM((CH,), jnp.int32),
                       pltpu.VMEM((CH,), jnp.int32),
                       pltpu.SemaphoreType.DMA,
                       pltpu.SemaphoreType.DMA,
                       pltpu.SemaphoreType.DMA,
                       pltpu.SemaphoreType.DMA],
    )
    def sc_scatter_rows(x_hbm, pos_hbm, xs_hbm,
                        rb0, rb1, pb0, pb1, sr0, sr1, sp0, sp1):
        wid = lax.axis_index("s") * SC_NC + lax.axis_index("c")
        base = wid * RPW
        rbufs, pbufs = (rb0, rb1), (pb0, pb1)
        rsems, psems = (sr0, sr1), (sp0, sp1)

        def start(ch):
            r0 = base + ch * CH
            s = ch % 2
            pltpu.async_copy(pos_hbm.at[pl.ds(r0, CH)], pbufs[s], psems[s])
            pltpu.async_copy(x_hbm.at[pl.ds(r0, CH)], rbufs[s], rsems[s])

        start(0)
        start(1)
        for ch in range(NCH):
            s = ch % 2
            r0 = base + ch * CH
            pltpu.make_async_copy(pos_hbm.at[pl.ds(r0, CH)], pbufs[s],
                                  psems[s]).wait()
            pltpu.make_async_copy(x_hbm.at[pl.ds(r0, CH)], rbufs[s],
                                  rsems[s]).wait()
            pltpu.async_copy(rbufs[s], xs_hbm.at[pbufs[s]], rsems[s]).wait()
            if ch + 2 < NCH:
                start(ch + 2)

    @functools.partial(
        pl.kernel,
        out_type=jax.ShapeDtypeStruct((L, D), jnp.float32),
        mesh=mesh,
        scratch_types=[pltpu.VMEM((CH, D), jnp.float32),
                       pltpu.VMEM((CH, D), jnp.float32),
                       pltpu.VMEM((CH,), jnp.int32),
                       pltpu.VMEM((CH,), jnp.int32),
                       pltpu.SemaphoreType.DMA,
                       pltpu.SemaphoreType.DMA,
                       pltpu.SemaphoreType.DMA,
                       pltpu.SemaphoreType.DMA],
    )
    def sc_gather_rows(y_hbm, pos_hbm, out_hbm,
                       rb0, rb1, pb0, pb1, sr0, sr1, sp0, sp1):
        wid = lax.axis_index("s") * SC_NC + lax.axis_index("c")
        base = wid * RPW
        rbufs, pbufs = (rb0, rb1), (pb0, pb1)
        rsems, psems = (sr0, sr1), (sp0, sp1)

        def start(ch):
            r0 = base + ch * CH
            s = ch % 2
            pltpu.async_copy(pos_hbm.at[pl.ds(r0, CH)], pbufs[s], psems[s])

        def fire_gather(ch):
            s = ch % 2
            r0 = base + ch * CH
            pltpu.make_async_copy(pos_hbm.at[pl.ds(r0, CH)], pbufs[s],
                                  psems[s]).wait()
            pltpu.async_copy(y_hbm.at[pbufs[s]], rbufs[s], rsems[s])

        start(0)
        start(1)
        fire_gather(0)
        for ch in range(NCH):
            s = ch % 2
            r0 = base + ch * CH
            if ch + 1 < NCH:
                fire_gather(ch + 1)
            pltpu.make_async_copy(y_hbm.at[pbufs[s]], rbufs[s], rsems[s]).wait()
            pltpu.sync_copy(rbufs[s], out_hbm.at[pl.ds(r0, CH)])
            if ch + 2 < NCH:
                start(ch + 2)

    return sc_scatter_rows, sc_gather_rows


def _hilo(v):
    hi = jnp.floor(v * (1.0 / 256.0))
    return hi, v - hi * 256.0


def _mm_exact(a, b, dims, split_b):
    f32 = jnp.float32

    def mm(x, y):
        return jax.lax.dot_general(x, y, dims, preferred_element_type=f32)

    if split_b:
        hi, lo = _hilo(b)
        return mm(a, hi) * 256.0 + mm(a, lo)
    hi, lo = _hilo(a)
    return mm(hi, b) * 256.0 + mm(lo, b)


def _onehot_eye(keys_ref):
    kf = keys_ref[...].reshape(1, BLK).astype(jnp.float32)
    kcol = _transpose_col(kf)
    lane = jax.lax.broadcasted_iota(jnp.int32, (1, 256), 1).astype(jnp.float32)
    onehot = (kcol == lane).astype(jnp.float32)
    eye_rows = jax.lax.broadcasted_iota(jnp.int32, (BLK, BLK), 0)
    eye_cols = jax.lax.broadcasted_iota(jnp.int32, (BLK, BLK), 1)
    return onehot, eye_rows, eye_cols


def _count_p0_kernel(keys_ref, pw_ref, offs_ref, hist_ref):
    b = pl.program_id(0)
    onehot, eye_rows, eye_cols = _onehot_eye(keys_ref)
    eye = (eye_rows == eye_cols).astype(jnp.float32)

    @pl.when(b == 0)
    def _init():
        hist_ref[...] = jnp.zeros((1, 256), jnp.float32)

    lt = (eye_rows > eye_cols).astype(jnp.bfloat16)
    csum = jnp.dot(lt, onehot.astype(jnp.bfloat16),
                   preferred_element_type=jnp.float32)
    rank = jnp.sum(onehot * csum, axis=1, keepdims=True)
    run = _mm_exact(onehot, hist_ref[...], (((1,), (1,)), ((), ())),
                    split_b=True)
    pw_col = rank + run
    pw_row = _mm_exact(pw_col, eye, (((0,), (0,)), ((), ())),
                       split_b=False)
    pw_ref[...] = pw_row.reshape(1, 1, BLK)
    hist_ref[...] += jnp.sum(onehot, axis=0, keepdims=True)

    @pl.when(b == NB - 1)
    def _offsets():
        c1 = jax.lax.broadcasted_iota(jnp.int32, (256, 256), 0)
        c2 = jax.lax.broadcasted_iota(jnp.int32, (256, 256), 1)
        strict = (c1 < c2).astype(jnp.float32)
        off = _mm_exact(hist_ref[...], strict, (((1,), (0,)), ((), ())),
                        split_b=False)
        offs_ref[...] = (off + 0.5).astype(jnp.int32)


def _count_p1_kernel(keys_ref, pw_ref, offs_ref, pos_ref, ks_ref):
    b = pl.program_id(0)
    onehot, eye_rows, eye_cols = _onehot_eye(keys_ref)
    eye = (eye_rows == eye_cols).astype(jnp.float32)
    run = _mm_exact(onehot, offs_ref[...].astype(jnp.float32),
                    (((1,), (1,)), ((), ())), split_b=True)
    run_row = _mm_exact(run, eye, (((0,), (0,)), ((), ())),
                        split_b=False)
    pos_ref[...] = (pw_ref[...].reshape(1, BLK)
                    + run_row + 0.5).astype(jnp.int32).reshape(1, 1, BLK)
    jcol = (b * BLK
            + jax.lax.broadcasted_iota(jnp.int32, (BLK, 1), 0))
    le = (offs_ref[...] <= jcol).astype(jnp.float32)
    scol = jnp.sum(le, axis=1, keepdims=True) - 1.0
    srow = _mm_exact(scol, eye, (((0,), (0,)), ((), ())), split_b=False)
    ks_ref[...] = (srow + 0.5).astype(jnp.int32).reshape(1, 1, BLK)


def _count_sort(keys3):
    blk_spec = pl.BlockSpec((1, 1, BLK), lambda i: (i, 0, 0))
    offs_spec = pl.BlockSpec((1, 256), lambda i: (0, 0))
    pw3, offs2 = pl.pallas_call(
        _count_p0_kernel,
        grid=(NB,),
        in_specs=[blk_spec],
        out_specs=[blk_spec, offs_spec],
        out_shape=[jax.ShapeDtypeStruct((NB, 1, BLK), jnp.float32),
                   jax.ShapeDtypeStruct((1, 256), jnp.int32)],
        scratch_shapes=[pltpu.VMEM((1, 256), jnp.float32)],
    )(keys3)
    pos3, ks3 = pl.pallas_call(
        _count_p1_kernel,
        grid=(NB,),
        in_specs=[blk_spec, blk_spec, offs_spec],
        out_specs=[blk_spec, blk_spec],
        out_shape=[jax.ShapeDtypeStruct((NB, 1, BLK), jnp.int32),
                   jax.ShapeDtypeStruct((NB, 1, BLK), jnp.int32)],
    )(keys3, pw3, offs2)
    return pos3, ks3, offs2


def _ln(x, g, b):
    mu = jnp.mean(x, axis=-1, keepdims=True)
    xc = x - mu
    var = jnp.mean(xc * xc, axis=-1, keepdims=True)
    return xc * jax.lax.rsqrt(var + EPS) * g + b


def _transpose_col(vec_1xn):
    n = vec_1xn.shape[1]
    rows = jax.lax.broadcasted_iota(jnp.int32, (n, n), 0)
    cols = jax.lax.broadcasted_iota(jnp.int32, (n, n), 1)
    eye = (rows == cols).astype(jnp.float32)
    return jax.lax.dot_general(eye, vec_1xn, (((1,), (1,)), ((), ())),
                               preferred_element_type=jnp.float32)


def _tx_kernel(nv_ref, lo_ref, hi_ref, x_any, xq_ref, keys_ref,
               wqkvT_ref, bqkv_ref, woT_ref, bo_ref,
               ln1g_ref, ln1b_ref, ln2g_ref, ln2b_ref,
               w1T_ref, b1_ref, w2T_ref, b2_ref, lnfg_ref, lnfb_ref,
               y_ref, xkv_ref, sems):
    qb = pl.program_id(0)
    nv = nv_ref[0]
    bstart = qb * BLK

    @pl.when(bstart >= nv)
    def _zero():
        y_ref[...] = jnp.zeros((BLK, D), jnp.float32)

    @pl.when(bstart < nv)
    def _compute():
        kb_lo = lo_ref[qb]
        kb_hi = hi_ref[qb]
        trip = kb_hi - kb_lo

        def kb_of(idx):
            base = kb_lo + idx
            return base + (base >= qb).astype(jnp.int32)

        def start_fetch(idx):
            slot = jax.lax.rem(idx, NBUF)
            pltpu.make_async_copy(
                x_any.at[pl.ds(kb_of(idx) * BLK, BLK)],
                xkv_ref.at[slot], sems.at[slot]).start()

        @pl.when(trip > 0)
        def _p0():
            start_fetch(0)

        @pl.when(trip > 1)
        def _p1():
            start_fetch(1)

        ln1g = ln1g_ref[...]
        ln1b = ln1b_ref[...]
        wqkvT = wqkvT_ref[...]
        bqkv = bqkv_ref[...]
        inv_scale = jnp.float32(1.0) / jnp.float32(DH) ** 0.5

        xq = xq_ref[...]
        ln1q = _ln(xq, ln1g, ln1b)
        qkv = jnp.dot(ln1q, wqkvT, preferred_element_type=jnp.float32) + bqkv
        q = qkv[:, :D] * inv_scale
        k_own = qkv[:, D:2 * D]
        v_own = qkv[:, 2 * D:]

        kq = keys_ref[qb].astype(jnp.float32)
        kq_col = _transpose_col(kq)

        mask_own = kq_col == kq
        carry = []
        for h in range(NH):
            qh = q[:, h * DH:(h + 1) * DH]
            kh = k_own[:, h * DH:(h + 1) * DH]
            vh = v_own[:, h * DH:(h + 1) * DH]
            logits = jax.lax.dot_general(qh, kh, (((1,), (1,)), ((), ())),
                                         preferred_element_type=jnp.float32)
            logits = jnp.where(mask_own, logits, NEG)
            m0 = jnp.max(logits, axis=1, keepdims=True)
            p = jnp.exp(logits - m0)
            l0 = jnp.sum(p, axis=1, keepdims=True)
            a0 = jnp.dot(p, vh, preferred_element_type=jnp.float32)
            carry += [m0, l0, a0]

        def body(idx, carry):
            slot = jax.lax.rem(idx, NBUF)
            kb = kb_of(idx)

            @pl.when(idx + 2 < trip)
            def _pf():
                start_fetch(idx + 2)

            pltpu.make_async_copy(
                x_any.at[pl.ds(kb * BLK, BLK)],
                xkv_ref.at[slot], sems.at[slot]).wait()
            xk = xkv_ref[slot]
            lnk = _ln(xk, ln1g, ln1b)
            kv = jnp.dot(lnk, wqkvT[:, D:],
                         preferred_element_type=jnp.float32) + bqkv[:, D:]
            k = kv[:, :D]
            v = kv[:, D:]
            kk = keys_ref[kb].astype(jnp.float32)
            mask = kq_col == kk
            new = []
            for h in range(NH):
                m_h, l_h, a_h = carry[3 * h], carry[3 * h + 1], carry[3 * h + 2]
                qh = q[:, h * DH:(h + 1) * DH]
                kh = k[:, h * DH:(h + 1) * DH]
                vh = v[:, h * DH:(h + 1) * DH]
                logits = jax.lax.dot_general(qh, kh, (((1,), (1,)), ((), ())),
                                             preferred_element_type=jnp.float32)
                logits = jnp.where(mask, logits, NEG)
                m_new = jnp.maximum(m_h, jnp.max(logits, axis=1, keepdims=True))
                alpha = jnp.exp(m_h - m_new)
                p = jnp.exp(logits - m_new)
                l_new = l_h * alpha + jnp.sum(p, axis=1, keepdims=True)
                a_new = a_h * alpha + jnp.dot(p, vh,
                                              preferred_element_type=jnp.float32)
                new += [m_new, l_new, a_new]
            return tuple(new)

        carry = jax.lax.fori_loop(0, trip, body, tuple(carry))

        attn = jnp.concatenate(
            [carry[3 * h + 2] / carry[3 * h + 1] for h in range(NH)], axis=1)
        proj = jnp.dot(attn, woT_ref[...],
                       preferred_element_type=jnp.float32) + bo_ref[...]
        x1 = xq + proj
        h2 = _ln(x1, ln2g_ref[...], ln2b_ref[...])
        ff = jnp.maximum(jnp.dot(h2, w1T_ref[...],
                                 preferred_element_type=jnp.float32)
                         + b1_ref[...], 0.0)
        ff = jnp.dot(ff, w2T_ref[...],
                     preferred_element_type=jnp.float32) + b2_ref[...]
        x2 = x1 + ff
        y = _ln(x2, lnfg_ref[...], lnfb_ref[...])
        rows = bstart + jax.lax.broadcasted_iota(jnp.int32, (BLK, 1), 0)
        y_ref[...] = jnp.where(rows < nv, y, 0.0)


def _run_transformer(xs, keys3, nv, kb_lo, kb_hi, wqkvT, bqkv, woT, bo,
                     ln1g, ln1b, ln2g, ln2b, w1T, b1, w2T, b2, lnfg, lnfb):
    smem = pl.BlockSpec(memory_space=pltpu.MemorySpace.SMEM)
    hbm = pl.BlockSpec(memory_space=pltpu.MemorySpace.HBM)

    def full(shape):
        nd = len(shape)
        return pl.BlockSpec(shape, lambda i, _n=nd: (0,) * _n)

    return pl.pallas_call(
        _tx_kernel,
        grid=(NB,),
        in_specs=[
            smem, smem, smem, hbm,
            pl.BlockSpec((BLK, D), lambda i: (i, 0)),
            full((NB, 1, BLK)),
            full((D, 3 * D)), full((1, 3 * D)), full((D, D)), full((1, D)),
            full((1, D)), full((1, D)), full((1, D)), full((1, D)),
            full((D, DFF)), full((1, DFF)), full((DFF, D)), full((1, D)),
            full((1, D)), full((1, D)),
        ],
        out_specs=pl.BlockSpec((BLK, D), lambda i: (i, 0)),
        out_shape=jax.ShapeDtypeStruct((L, D), jnp.float32),
        scratch_shapes=[pltpu.VMEM((NBUF, BLK, D), jnp.float32),
                        pltpu.SemaphoreType.DMA((NBUF,))],
    )(nv, kb_lo, kb_hi, xs, xs, keys3, wqkvT, bqkv, woT, bo,
      ln1g, ln1b, ln2g, ln2b, w1T, b1, w2T, b2, lnfg, lnfb)


def kernel(rel_tokens_all, pe, Wqkv, bqkv, Wo, bo, ln1_g, ln1_b, ln2_g, ln2_b,
           W1, b1, W2, b2, lnf_g, lnf_b, pair_valid, padded_pidx, padded_oidx):
    keys = (padded_pidx.astype(jnp.int32) * 16
            + padded_oidx.astype(jnp.int32)).reshape(-1)
    valid = pair_valid.reshape(-1)
    keys = jnp.where(valid, keys, NKEY)

    pos3, ks3, offs2 = _count_sort(keys.reshape(NB, 1, BLK))
    pos = pos3.reshape(L)
    offsets = offs2.reshape(256)
    n_valid = offsets[NKEY:NKEY + 1]

    x = (rel_tokens_all + pe[:T][:, None, :]).reshape(L, D)
    sc_scatter_rows, _ = _sc_kernels()
    xs = sc_scatter_rows(x, pos)
    ks2 = ks3.reshape(NB, BLK)
    kfirst = ks2[:, 0]
    klast = ks2[:, -1]
    bstart = jnp.arange(NB, dtype=jnp.int32) * BLK
    lo = jnp.where(kfirst < NKEY, offsets[kfirst], bstart)
    hi = jnp.where(klast < NKEY, offsets[klast + 1], bstart + BLK)
    kb_lo = lo // BLK
    kb_hi = (hi - 1) // BLK

    keys3 = ks3

    y = _run_transformer(
        xs, keys3, n_valid, kb_lo, kb_hi,
        Wqkv.T, bqkv.reshape(1, -1), Wo.T, bo.reshape(1, -1),
        ln1_g.reshape(1, -1), ln1_b.reshape(1, -1),
        ln2_g.reshape(1, -1), ln2_b.reshape(1, -1),
        W1.T, b1.reshape(1, -1), W2.T, b2.reshape(1, -1),
        lnf_g.reshape(1, -1), lnf_b.reshape(1, -1))

    return _sc_kernels()[1](y, pos).reshape(T, K_MAX, D)


# speedup vs baseline: 1.0547x; 1.0547x over previous
import functools

import jax
import jax.numpy as jnp
from jax import lax
from jax.experimental import pallas as pl
from jax.experimental.pallas import tpu as pltpu
from jax.experimental.pallas import tpu_sc as plsc

T, K_MAX, D = 256, 128, 128
L = T * K_MAX
NH, DH = 4, 32
DFF = 256
BLK = 256
NB = L // BLK
NKEY = 128
EPS = 1e-5
NEG = -1e9
NBUF = 3


SC_NC, SC_NS = 2, 16
SC_NW = SC_NC * SC_NS
RPW = L // SC_NW
CH = 128
NCH = RPW // CH

@functools.cache
def _sc_kernels():
    mesh = plsc.VectorSubcoreMesh(core_axis_name="c", subcore_axis_name="s")

    @functools.partial(
        pl.kernel,
        out_type=jax.ShapeDtypeStruct((L, D), jnp.float32),
        mesh=mesh,
        scratch_types=[pltpu.VMEM((CH, D), jnp.float32),
                       pltpu.VMEM((CH, D), jnp.float32),
                       pltpu.VMEM((CH,), jnp.int32),
                       pltpu.VMEM((CH,), jnp.int32),
                       pltpu.SemaphoreType.DMA,
                       pltpu.SemaphoreType.DMA,
                       pltpu.SemaphoreType.DMA,
                       pltpu.SemaphoreType.DMA],
    )
    def sc_scatter_rows(x_hbm, pos_hbm, xs_hbm,
                        rb0, rb1, pb0, pb1, sr0, sr1, sp0, sp1):
        wid = lax.axis_index("s") * SC_NC + lax.axis_index("c")
        base = wid * RPW
        rbufs, pbufs = (rb0, rb1), (pb0, pb1)
        rsems, psems = (sr0, sr1), (sp0, sp1)

        def start(ch):
            r0 = base + ch * CH
            s = ch % 2
            pltpu.async_copy(pos_hbm.at[pl.ds(r0, CH)], pbufs[s], psems[s])
            pltpu.async_copy(x_hbm.at[pl.ds(r0, CH)], rbufs[s], rsems[s])

        start(0)
        start(1)
        for ch in range(NCH):
            s = ch % 2
            r0 = base + ch * CH
            pltpu.make_async_copy(pos_hbm.at[pl.ds(r0, CH)], pbufs[s],
                                  psems[s]).wait()
            pltpu.make_async_copy(x_hbm.at[pl.ds(r0, CH)], rbufs[s],
                                  rsems[s]).wait()
            pltpu.async_copy(rbufs[s], xs_hbm.at[pbufs[s]], rsems[s]).wait()
            if ch + 2 < NCH:
                start(ch + 2)

    @functools.partial(
        pl.kernel,
        out_type=jax.ShapeDtypeStruct((L, D), jnp.float32),
        mesh=mesh,
        scratch_types=[pltpu.VMEM((CH, D), jnp.float32),
                       pltpu.VMEM((CH, D), jnp.float32),
                       pltpu.VMEM((CH,), jnp.int32),
                       pltpu.VMEM((CH,), jnp.int32),
                       pltpu.SemaphoreType.DMA,
                       pltpu.SemaphoreType.DMA,
                       pltpu.SemaphoreType.DMA,
                       pltpu.SemaphoreType.DMA],
    )
    def sc_gather_rows(y_hbm, pos_hbm, out_hbm,
                       rb0, rb1, pb0, pb1, sr0, sr1, sp0, sp1):
        wid = lax.axis_index("s") * SC_NC + lax.axis_index("c")
        base = wid * RPW
        rbufs, pbufs = (rb0, rb1), (pb0, pb1)
        rsems, psems = (sr0, sr1), (sp0, sp1)

        def start(ch):
            r0 = base + ch * CH
            s = ch % 2
            pltpu.async_copy(pos_hbm.at[pl.ds(r0, CH)], pbufs[s], psems[s])

        def fire_gather(ch):
            s = ch % 2
            r0 = base + ch * CH
            pltpu.make_async_copy(pos_hbm.at[pl.ds(r0, CH)], pbufs[s],
                                  psems[s]).wait()
            pltpu.async_copy(y_hbm.at[pbufs[s]], rbufs[s], rsems[s])

        start(0)
        start(1)
        fire_gather(0)
        for ch in range(NCH):
            s = ch % 2
            r0 = base + ch * CH
            if ch + 1 < NCH:
                fire_gather(ch + 1)
            pltpu.make_async_copy(y_hbm.at[pbufs[s]], rbufs[s], rsems[s]).wait()
            pltpu.sync_copy(rbufs[s], out_hbm.at[pl.ds(r0, CH)])
            if ch + 2 < NCH:
                start(ch + 2)

    return sc_scatter_rows, sc_gather_rows


def _hilo(v):
    hi = jnp.floor(v * (1.0 / 256.0))
    return hi, v - hi * 256.0


def _mm_exact(a, b, dims, split_b):
    f32 = jnp.float32

    def mm(x, y):
        return jax.lax.dot_general(x, y, dims, preferred_element_type=f32)

    if split_b:
        hi, lo = _hilo(b)
        return mm(a, hi) * 256.0 + mm(a, lo)
    hi, lo = _hilo(a)
    return mm(hi, b) * 256.0 + mm(lo, b)


def _onehot_t(keys_ref):
    krow = keys_ref[...].reshape(1, BLK).astype(jnp.float32)
    crow = jax.lax.broadcasted_iota(jnp.int32, (256, BLK), 0).astype(jnp.float32)
    return (crow == krow).astype(jnp.float32)


def _count_p0_kernel(keys_ref, pw_ref, offs_ref, hist_ref, offacc_ref):
    b = pl.program_id(0)
    krow = keys_ref[...].reshape(1, BLK).astype(jnp.float32)
    crow = (jax.lax.broadcasted_iota(jnp.int32, (256, BLK), 0)
            .astype(jnp.float32))
    oht = (crow == krow).astype(jnp.float32)

    @pl.when(b == 0)
    def _init():
        hist_ref[...] = jnp.zeros((256, 1), jnp.float32)
        offacc_ref[...] = jnp.zeros((256, 1), jnp.float32)

    r1 = jax.lax.broadcasted_iota(jnp.int32, (BLK, BLK), 0)
    c1 = jax.lax.broadcasted_iota(jnp.int32, (BLK, BLK), 1)
    ut = (r1 < c1).astype(jnp.bfloat16)
    csum_t = jax.lax.dot_general(oht.astype(jnp.bfloat16), ut,
                                 (((1,), (0,)), ((), ())),
                                 preferred_element_type=jnp.float32)
    rank_row = jnp.sum(oht * csum_t, axis=0, keepdims=True)
    run_row = jnp.sum(oht * hist_ref[...], axis=0, keepdims=True)
    pw_ref[...] = (rank_row + run_row).reshape(1, 1, BLK)
    hist_ref[...] += jnp.sum(oht, axis=1, keepdims=True)
    offacc_ref[...] += jnp.sum((krow < crow).astype(jnp.float32),
                               axis=1, keepdims=True)

    @pl.when(b == NB - 1)
    def _offsets():
        offs_ref[...] = (offacc_ref[...] + 0.5).astype(jnp.int32)


def _count_p1_kernel(keys_ref, pw_ref, offs_ref, pos_ref, ks_ref):
    b = pl.program_id(0)
    oht = _onehot_t(keys_ref)
    offs_col = offs_ref[...].astype(jnp.float32)
    run_row = jnp.sum(oht * offs_col, axis=0, keepdims=True)
    pos_ref[...] = (pw_ref[...].reshape(1, BLK)
                    + run_row + 0.5).astype(jnp.int32).reshape(1, 1, BLK)
    jrow = (b * BLK
            + jax.lax.broadcasted_iota(jnp.int32, (1, BLK), 1))
    le = (offs_ref[...] <= jrow).astype(jnp.float32)
    ks = jnp.sum(le, axis=0, keepdims=True) - 1.0
    ks_ref[...] = (ks + 0.5).astype(jnp.int32).reshape(1, 1, BLK)


def _count_sort(keys3):
    blk_spec = pl.BlockSpec((1, 1, BLK), lambda i: (i, 0, 0))
    offs_spec = pl.BlockSpec((256, 1), lambda i: (0, 0))
    pw3, offsc = pl.pallas_call(
        _count_p0_kernel,
        grid=(NB,),
        in_specs=[blk_spec],
        out_specs=[blk_spec, offs_spec],
        out_shape=[jax.ShapeDtypeStruct((NB, 1, BLK), jnp.float32),
                   jax.ShapeDtypeStruct((256, 1), jnp.int32)],
        scratch_shapes=[pltpu.VMEM((256, 1), jnp.float32),
                        pltpu.VMEM((256, 1), jnp.float32)],
    )(keys3)
    pos3, ks3 = pl.pallas_call(
        _count_p1_kernel,
        grid=(NB,),
        in_specs=[blk_spec, blk_spec, offs_spec],
        out_specs=[blk_spec, blk_spec],
        out_shape=[jax.ShapeDtypeStruct((NB, 1, BLK), jnp.int32),
                   jax.ShapeDtypeStruct((NB, 1, BLK), jnp.int32)],
    )(keys3, pw3, offsc)
    return pos3, ks3, offsc


def _ln(x, g, b):
    mu = jnp.mean(x, axis=-1, keepdims=True)
    xc = x - mu
    var = jnp.mean(xc * xc, axis=-1, keepdims=True)
    return xc * jax.lax.rsqrt(var + EPS) * g + b


def _transpose_col(vec_1xn):
    n = vec_1xn.shape[1]
    rows = jax.lax.broadcasted_iota(jnp.int32, (n, n), 0)
    cols = jax.lax.broadcasted_iota(jnp.int32, (n, n), 1)
    eye = (rows == cols).astype(jnp.float32)
    return jax.lax.dot_general(eye, vec_1xn, (((1,), (1,)), ((), ())),
                               preferred_element_type=jnp.float32)


def _tx_kernel(nv_ref, lo_ref, hi_ref, x_any, xq_ref, keys_ref,
               wqkvT_ref, bqkv_ref, woT_ref, bo_ref,
               ln1g_ref, ln1b_ref, ln2g_ref, ln2b_ref,
               w1T_ref, b1_ref, w2T_ref, b2_ref, lnfg_ref, lnfb_ref,
               y_ref, xkv_ref, sems):
    qb = pl.program_id(0)
    nv = nv_ref[0]
    bstart = qb * BLK

    @pl.when(bstart >= nv)
    def _zero():
        y_ref[...] = jnp.zeros((BLK, D), jnp.float32)

    @pl.when(bstart < nv)
    def _compute():
        kb_lo = lo_ref[qb]
        kb_hi = hi_ref[qb]
        trip = kb_hi - kb_lo

        def kb_of(idx):
            base = kb_lo + idx
            return base + (base >= qb).astype(jnp.int32)

        def start_fetch(idx):
            slot = jax.lax.rem(idx, NBUF)
            pltpu.make_async_copy(
                x_any.at[pl.ds(kb_of(idx) * BLK, BLK)],
                xkv_ref.at[slot], sems.at[slot]).start()

        @pl.when(trip > 0)
        def _p0():
            start_fetch(0)

        @pl.when(trip > 1)
        def _p1():
            start_fetch(1)

        ln1g = ln1g_ref[...]
        ln1b = ln1b_ref[...]
        wqkvT = wqkvT_ref[...]
        bqkv = bqkv_ref[...]
        inv_scale = jnp.float32(1.0) / jnp.float32(DH) ** 0.5

        xq = xq_ref[...]
        ln1q = _ln(xq, ln1g, ln1b)
        qkv = jnp.dot(ln1q, wqkvT, preferred_element_type=jnp.float32) + bqkv
        q = qkv[:, :D] * inv_scale
        k_own = qkv[:, D:2 * D]
        v_own = qkv[:, 2 * D:]

        kq = keys_ref[qb].astype(jnp.float32)
        kq_col = _transpose_col(kq)

        mask_own = kq_col == kq
        carry = []
        for h in range(NH):
            qh = q[:, h * DH:(h + 1) * DH]
            kh = k_own[:, h * DH:(h + 1) * DH]
            vh = v_own[:, h * DH:(h + 1) * DH]
            logits = jax.lax.dot_general(qh, kh, (((1,), (1,)), ((), ())),
                                         preferred_element_type=jnp.float32)
            logits = jnp.where(mask_own, logits, NEG)
            m0 = jnp.max(logits, axis=1, keepdims=True)
            p = jnp.exp(logits - m0)
            l0 = jnp.sum(p, axis=1, keepdims=True)
            a0 = jnp.dot(p, vh, preferred_element_type=jnp.float32)
            carry += [m0, l0, a0]

        def body(idx, carry):
            slot = jax.lax.rem(idx, NBUF)
            kb = kb_of(idx)

            @pl.when(idx + 2 < trip)
            def _pf():
                start_fetch(idx + 2)

            pltpu.make_async_copy(
                x_any.at[pl.ds(kb * BLK, BLK)],
                xkv_ref.at[slot], sems.at[slot]).wait()
            xk = xkv_ref[slot]
            lnk = _ln(xk, ln1g, ln1b)
            kv = jnp.dot(lnk, wqkvT[:, D:],
                         preferred_element_type=jnp.float32) + bqkv[:, D:]
            k = kv[:, :D]
            v = kv[:, D:]
            kk = keys_ref[kb].astype(jnp.float32)
            mask = kq_col == kk
            new = []
            for h in range(NH):
                m_h, l_h, a_h = carry[3 * h], carry[3 * h + 1], carry[3 * h + 2]
                qh = q[:, h * DH:(h + 1) * DH]
                kh = k[:, h * DH:(h + 1) * DH]
                vh = v[:, h * DH:(h + 1) * DH]
                logits = jax.lax.dot_general(qh, kh, (((1,), (1,)), ((), ())),
                                             preferred_element_type=jnp.float32)
                logits = jnp.where(mask, logits, NEG)
                m_new = jnp.maximum(m_h, jnp.max(logits, axis=1, keepdims=True))
                alpha = jnp.exp(m_h - m_new)
                p = jnp.exp(logits - m_new)
                l_new = l_h * alpha + jnp.sum(p, axis=1, keepdims=True)
                a_new = a_h * alpha + jnp.dot(p, vh,
                                              preferred_element_type=jnp.float32)
                new += [m_new, l_new, a_new]
            return tuple(new)

        carry = jax.lax.fori_loop(0, trip, body, tuple(carry))

        attn = jnp.concatenate(
            [carry[3 * h + 2] / carry[3 * h + 1] for h in range(NH)], axis=1)
        proj = jnp.dot(attn, woT_ref[...],
                       preferred_element_type=jnp.float32) + bo_ref[...]
        x1 = xq + proj
        h2 = _ln(x1, ln2g_ref[...], ln2b_ref[...])
        ff = jnp.maximum(jnp.dot(h2, w1T_ref[...],
                                 preferred_element_type=jnp.float32)
                         + b1_ref[...], 0.0)
        ff = jnp.dot(ff, w2T_ref[...],
                     preferred_element_type=jnp.float32) + b2_ref[...]
        x2 = x1 + ff
        y = _ln(x2, lnfg_ref[...], lnfb_ref[...])
        rows = bstart + jax.lax.broadcasted_iota(jnp.int32, (BLK, 1), 0)
        y_ref[...] = jnp.where(rows < nv, y, 0.0)


def _run_transformer(xs, keys3, nv, kb_lo, kb_hi, wqkvT, bqkv, woT, bo,
                     ln1g, ln1b, ln2g, ln2b, w1T, b1, w2T, b2, lnfg, lnfb):
    smem = pl.BlockSpec(memory_space=pltpu.MemorySpace.SMEM)
    hbm = pl.BlockSpec(memory_space=pltpu.MemorySpace.HBM)

    def full(shape):
        nd = len(shape)
        return pl.BlockSpec(shape, lambda i, _n=nd: (0,) * _n)

    return pl.pallas_call(
        _tx_kernel,
        grid=(NB,),
        in_specs=[
            smem, smem, smem, hbm,
            pl.BlockSpec((BLK, D), lambda i: (i, 0)),
            full((NB, 1, BLK)),
            full((D, 3 * D)), full((1, 3 * D)), full((D, D)), full((1, D)),
            full((1, D)), full((1, D)), full((1, D)), full((1, D)),
            full((D, DFF)), full((1, DFF)), full((DFF, D)), full((1, D)),
            full((1, D)), full((1, D)),
        ],
        out_specs=pl.BlockSpec((BLK, D), lambda i: (i, 0)),
        out_shape=jax.ShapeDtypeStruct((L, D), jnp.float32),
        scratch_shapes=[pltpu.VMEM((NBUF, BLK, D), jnp.float32),
                        pltpu.SemaphoreType.DMA((NBUF,))],
    )(nv, kb_lo, kb_hi, xs, xs, keys3, wqkvT, bqkv, woT, bo,
      ln1g, ln1b, ln2g, ln2b, w1T, b1, w2T, b2, lnfg, lnfb)


def kernel(rel_tokens_all, pe, Wqkv, bqkv, Wo, bo, ln1_g, ln1_b, ln2_g, ln2_b,
           W1, b1, W2, b2, lnf_g, lnf_b, pair_valid, padded_pidx, padded_oidx):
    keys = (padded_pidx.astype(jnp.int32) * 16
            + padded_oidx.astype(jnp.int32)).reshape(-1)
    valid = pair_valid.reshape(-1)
    keys = jnp.where(valid, keys, NKEY)

    pos3, ks3, offs2 = _count_sort(keys.reshape(NB, 1, BLK))
    pos = pos3.reshape(L)
    offsets = offs2.reshape(256)
    n_valid = offsets[NKEY:NKEY + 1]

    x = (rel_tokens_all + pe[:T][:, None, :]).reshape(L, D)
    sc_scatter_rows, _ = _sc_kernels()
    xs = sc_scatter_rows(x, pos)
    ks2 = ks3.reshape(NB, BLK)
    kfirst = ks2[:, 0]
    klast = ks2[:, -1]
    bstart = jnp.arange(NB, dtype=jnp.int32) * BLK
    lo = jnp.where(kfirst < NKEY, offsets[kfirst], bstart)
    hi = jnp.where(klast < NKEY, offsets[klast + 1], bstart + BLK)
    kb_lo = lo // BLK
    kb_hi = (hi - 1) // BLK

    keys3 = ks3

    y = _run_transformer(
        xs, keys3, n_valid, kb_lo, kb_hi,
        Wqkv.T, bqkv.reshape(1, -1), Wo.T, bo.reshape(1, -1),
        ln1_g.reshape(1, -1), ln1_b.reshape(1, -1),
        ln2_g.reshape(1, -1), ln2_b.reshape(1, -1),
        W1.T, b1.reshape(1, -1), W2.T, b2.reshape(1, -1),
        lnf_g.reshape(1, -1), lnf_b.reshape(1, -1))

    return _sc_kernels()[1](y, pos).reshape(T, K_MAX, D)
